# Initial kernel scaffold; baseline (speedup 1.0000x reference)
import jax
import jax.numpy as jnp
from jax.experimental import pallas as pl


def kernel(images):
    raise NotImplementedError("write your pallas kernel here")


# speedup vs baseline: 13.2336x; 13.2336x over previous
import numpy as np
import jax
import jax.numpy as jnp
from jax.experimental import pallas as pl
from jax.experimental.pallas import tpu as pltpu

_RATIO = 0.6
_ROT_FACTOR = 0.1


def _mask_params(B, H, W):
    rng = np.random.default_rng(0)
    lo = int(min(H * 0.5, W * 0.3))
    hi = int(max(H * 0.5, W * 0.3)) + 1
    ms = int(2 * max(H, W))
    rows = []
    for _ in range(B):
        g = int(rng.integers(lo, hi))
        if _RATIO == 1:
            l = int(rng.integers(1, g + 1))
        else:
            l = int(min(max(int(g * _RATIO + 0.5), 1), g - 1))
        s1 = int(rng.integers(0, g + 1))
        s2 = int(rng.integers(0, g + 1))
        ang = float(rng.uniform(-_ROT_FACTOR * 2.0 * np.pi,
                                _ROT_FACTOR * 2.0 * np.pi))
        n = ms // g
        rows.append([np.cos(ang), np.sin(ang), float(g), float(l),
                     float(n), float(s1), float(s2), 0.0])
    return np.asarray(rows, dtype=np.float32)


def _stripe(jf, sf, gf, lf, nf):
    t = jnp.maximum(jf - sf, 0.0)
    k0 = jnp.floor(t / gf + 0.5)
    rem0 = t - k0 * gf
    neg = (rem0 < 0.0).astype(jnp.float32)
    k = k0 - neg
    rem = rem0 + neg * gf
    ok = (jf >= sf) & (rem < lf) & (k < nf)
    return ok.astype(jnp.float32)


def _body(RY, WC, C, cy, offh, offw, p_ref, x_ref, o_ref):
    b = pl.program_id(0)
    r = pl.program_id(1)
    ca = p_ref[b, 0]
    sa = p_ref[b, 1]
    gf = p_ref[b, 2]
    lf = p_ref[b, 3]
    nf = p_ref[b, 4]
    s1 = p_ref[b, 5]
    s2 = p_ref[b, 6]

    yrow = jax.lax.broadcasted_iota(jnp.int32, (RY, WC), 0).astype(jnp.float32)
    elane = jax.lax.broadcasted_iota(jnp.int32, (RY, WC), 1).astype(jnp.float32)
    yf = yrow + ((r * RY + offh) - cy).astype(jnp.float32)
    xf = jnp.floor(elane * jnp.float32(1.0 / C)) + jnp.float32(offw - cy)

    ys = ca * yf + sa * xf + cy
    xs = ca * xf - sa * yf + cy
    y0 = jnp.floor(ys)
    x0 = jnp.floor(xs)
    fy = ys - y0
    fx = xs - x0
    r0 = _stripe(y0, s1, gf, lf, nf)
    r1 = _stripe(y0 + 1.0, s1, gf, lf, nf)
    c0 = _stripe(x0, s2, gf, lf, nf)
    c1 = _stripe(x0 + 1.0, s2, gf, lf, nf)
    m0 = jnp.maximum(r0, c0) * (1.0 - fx) + jnp.maximum(r0, c1) * fx
    m1 = jnp.maximum(r1, c0) * (1.0 - fx) + jnp.maximum(r1, c1) * fx
    m = m0 * (1.0 - fy) + m1 * fy
    o_ref[...] = x_ref[...] * m[None]


def kernel(images):
    B, H, W, C = images.shape
    ms = int(2 * max(H, W))
    cy = np.float32((ms - 1) / 2.0)
    offh = (ms - H) // 2
    offw = (ms - W) // 2
    params = jnp.asarray(_mask_params(B, H, W))

    WC = W * C
    RY = 64
    x = images.reshape(B, H, WC)

    import functools
    body = functools.partial(_body, RY, WC, C, cy, offh, offw)
    out = pl.pallas_call(
        body,
        grid=(B, H // RY),
        in_specs=[
            pl.BlockSpec(memory_space=pltpu.SMEM),
            pl.BlockSpec((1, RY, WC), lambda b, r: (b, r, 0)),
        ],
        out_specs=pl.BlockSpec((1, RY, WC), lambda b, r: (b, r, 0)),
        out_shape=jax.ShapeDtypeStruct((B, H, WC), images.dtype),
    )(params, x)
    return out.reshape(B, H, W, C)


# speedup vs baseline: 13.2454x; 1.0009x over previous
import numpy as np
import jax
import jax.numpy as jnp
from jax.experimental import pallas as pl
from jax.experimental.pallas import tpu as pltpu

_RATIO = 0.6
_ROT_FACTOR = 0.1


def _mask_params(B, H, W):
    rng = np.random.default_rng(0)
    lo = int(min(H * 0.5, W * 0.3))
    hi = int(max(H * 0.5, W * 0.3)) + 1
    ms = int(2 * max(H, W))
    rows = []
    for _ in range(B):
        g = int(rng.integers(lo, hi))
        if _RATIO == 1:
            l = int(rng.integers(1, g + 1))
        else:
            l = int(min(max(int(g * _RATIO + 0.5), 1), g - 1))
        s1 = int(rng.integers(0, g + 1))
        s2 = int(rng.integers(0, g + 1))
        ang = float(rng.uniform(-_ROT_FACTOR * 2.0 * np.pi,
                                _ROT_FACTOR * 2.0 * np.pi))
        n = ms // g
        rows.append([np.cos(ang), np.sin(ang), float(g), float(l),
                     float(n), float(s1), float(s2), 1.0 / g])
    return np.asarray(rows, dtype=np.float32)


def _stripe(jf, sf, gf, lf, nf, rg):
    t = jnp.maximum(jf - sf, 0.0)
    k0 = jnp.floor(t * rg + 0.5)
    rem0 = t - k0 * gf
    neg = (rem0 < 0.0).astype(jnp.float32)
    k = k0 - neg
    rem = rem0 + neg * gf
    ok = (jf >= sf) & (rem < lf) & (k < nf)
    return ok.astype(jnp.float32)


def _body(RY, WC, C, cy, offh, offw, p_ref, x_ref, o_ref):
    b = pl.program_id(0)
    r = pl.program_id(1)
    ca = p_ref[b, 0]
    sa = p_ref[b, 1]
    gf = p_ref[b, 2]
    lf = p_ref[b, 3]
    nf = p_ref[b, 4]
    s1 = p_ref[b, 5]
    s2 = p_ref[b, 6]
    rg = p_ref[b, 7]

    yrow = jax.lax.broadcasted_iota(jnp.int32, (RY, WC), 0).astype(jnp.float32)
    elane = jax.lax.broadcasted_iota(jnp.int32, (RY, WC), 1).astype(jnp.float32)
    yf = yrow + ((r * RY + offh) - cy).astype(jnp.float32)
    xf = jnp.floor(elane * jnp.float32(1.0 / C)) + jnp.float32(offw - cy)

    ys = ca * yf + sa * xf + cy
    xs = ca * xf - sa * yf + cy
    y0 = jnp.floor(ys)
    x0 = jnp.floor(xs)
    fy = ys - y0
    fx = xs - x0
    r0 = _stripe(y0, s1, gf, lf, nf, rg)
    r1 = _stripe(y0 + 1.0, s1, gf, lf, nf, rg)
    c0 = _stripe(x0, s2, gf, lf, nf, rg)
    c1 = _stripe(x0 + 1.0, s2, gf, lf, nf, rg)
    m0 = jnp.maximum(r0, c0) * (1.0 - fx) + jnp.maximum(r0, c1) * fx
    m1 = jnp.maximum(r1, c0) * (1.0 - fx) + jnp.maximum(r1, c1) * fx
    m = m0 * (1.0 - fy) + m1 * fy
    o_ref[...] = x_ref[...] * m[None]


def kernel(images):
    B, H, W, C = images.shape
    ms = int(2 * max(H, W))
    cy = np.float32((ms - 1) / 2.0)
    offh = (ms - H) // 2
    offw = (ms - W) // 2
    params = jnp.asarray(_mask_params(B, H, W))

    WC = W * C
    RY = 64
    x = images.reshape(B, H, WC)

    import functools
    body = functools.partial(_body, RY, WC, C, cy, offh, offw)
    out = pl.pallas_call(
        body,
        grid=(B, H // RY),
        in_specs=[
            pl.BlockSpec(memory_space=pltpu.SMEM),
            pl.BlockSpec((1, RY, WC), lambda b, r: (b, r, 0)),
        ],
        out_specs=pl.BlockSpec((1, RY, WC), lambda b, r: (b, r, 0)),
        out_shape=jax.ShapeDtypeStruct((B, H, WC), images.dtype),
    )(params, x)
    return out.reshape(B, H, W, C)


# speedup vs baseline: 40.3517x; 3.0465x over previous
import numpy as np
import jax
import jax.numpy as jnp
from jax.experimental import pallas as pl
from jax.experimental.pallas import tpu as pltpu

_RATIO = 0.6
_ROT_FACTOR = 0.1


def _mask_params(B, H, W):
    rng = np.random.default_rng(0)
    lo = int(min(H * 0.5, W * 0.3))
    hi = int(max(H * 0.5, W * 0.3)) + 1
    ms = int(2 * max(H, W))
    rows = []
    for _ in range(B):
        g = int(rng.integers(lo, hi))
        if _RATIO == 1:
            l = int(rng.integers(1, g + 1))
        else:
            l = int(min(max(int(g * _RATIO + 0.5), 1), g - 1))
        s1 = int(rng.integers(0, g + 1))
        s2 = int(rng.integers(0, g + 1))
        ang = float(rng.uniform(-_ROT_FACTOR * 2.0 * np.pi,
                                _ROT_FACTOR * 2.0 * np.pi))
        n = ms // g
        rows.append([np.cos(ang), np.sin(ang), float(g), float(l),
                     float(n), float(s1), float(s2), 1.0 / g])
    return np.asarray(rows, dtype=np.float32)


def _stripe(jf, sf, gf, lf, nf, rg):
    t = jnp.maximum(jf - sf, 0.0)
    k0 = jnp.floor(t * rg + 0.5)
    rem0 = t - k0 * gf
    neg = (rem0 < 0.0).astype(jnp.float32)
    k = k0 - neg
    rem = rem0 + neg * gf
    ok = (jf >= sf) & (rem < lf) & (k < nf)
    return ok.astype(jnp.float32)


def _body(RY, W, C, cy, offh, offw, p_ref, x_ref, o_ref):
    b = pl.program_id(0)
    r = pl.program_id(1)
    ca = p_ref[b, 0]
    sa = p_ref[b, 1]
    gf = p_ref[b, 2]
    lf = p_ref[b, 3]
    nf = p_ref[b, 4]
    s1 = p_ref[b, 5]
    s2 = p_ref[b, 6]
    rg = p_ref[b, 7]

    yrow = jax.lax.broadcasted_iota(jnp.int32, (RY, W), 0).astype(jnp.float32)
    xlane = jax.lax.broadcasted_iota(jnp.int32, (RY, W), 1).astype(jnp.float32)
    yf = yrow + ((r * RY + offh) - cy).astype(jnp.float32)
    xf = xlane + jnp.float32(offw - cy)

    ys = ca * yf + sa * xf + cy
    xs = ca * xf - sa * yf + cy
    y0 = jnp.floor(ys)
    x0 = jnp.floor(xs)
    fy = ys - y0
    fx = xs - x0
    r0 = _stripe(y0, s1, gf, lf, nf, rg)
    r1 = _stripe(y0 + 1.0, s1, gf, lf, nf, rg)
    c0 = _stripe(x0, s2, gf, lf, nf, rg)
    c1 = _stripe(x0 + 1.0, s2, gf, lf, nf, rg)
    m0 = jnp.maximum(r0, c0) * (1.0 - fx) + jnp.maximum(r0, c1) * fx
    m1 = jnp.maximum(r1, c0) * (1.0 - fx) + jnp.maximum(r1, c1) * fx
    m = m0 * (1.0 - fy) + m1 * fy
    for c in range(C):
        o_ref[0, c] = x_ref[0, c] * m


def kernel(images):
    B, H, W, C = images.shape
    ms = int(2 * max(H, W))
    cy = np.float32((ms - 1) / 2.0)
    offh = (ms - H) // 2
    offw = (ms - W) // 2
    params = jnp.asarray(_mask_params(B, H, W))

    RY = 64
    x = jnp.transpose(images, (0, 3, 1, 2))

    import functools
    body = functools.partial(_body, RY, W, C, cy, offh, offw)
    out = pl.pallas_call(
        body,
        grid=(B, H // RY),
        in_specs=[
            pl.BlockSpec(memory_space=pltpu.SMEM),
            pl.BlockSpec((1, C, RY, W), lambda b, r: (b, 0, r, 0)),
        ],
        out_specs=pl.BlockSpec((1, C, RY, W), lambda b, r: (b, 0, r, 0)),
        out_shape=jax.ShapeDtypeStruct((B, C, H, W), images.dtype),
    )(params, x)
    return jnp.transpose(out, (0, 2, 3, 1))
